# baseline (speedup 1.0000x reference)
import jax
import jax.numpy as jnp
from jax.experimental import pallas as pl
from jax.experimental.pallas import tpu as pltpu

_B, _N, _H = 1024, 50, 128
_ACT = 19
_BB = 4
_R = _BB * _N


def _fused(obs_ref, rnn_ref, masks_ref, W_obs_ref, b_obs_ref, W_ih_ref,
           W_hh_ref, b_gates_ref, W_me_ref, b_me_ref, W_g1_ref, a_g1_ref,
           W_g2_ref, a_g2_ref, W_md_ref, b_md_ref, W_act_ref, b_act_ref,
           out_ref):
    f32 = jnp.float32

    x = jnp.tanh(jnp.dot(obs_ref[...], W_obs_ref[...],
                         preferred_element_type=f32) + b_obs_ref[...])
    m = masks_ref[...]
    h0 = rnn_ref[:, :_H] * m
    c0 = rnn_ref[:, _H:] * m
    gates = (jnp.dot(x, W_ih_ref[...], preferred_element_type=f32)
             + jnp.dot(h0, W_hh_ref[...], preferred_element_type=f32)
             + b_gates_ref[...])
    ig = jax.nn.sigmoid(gates[:, :_H])
    fg = jax.nn.sigmoid(gates[:, _H:2 * _H])
    gg = jnp.tanh(gates[:, 2 * _H:3 * _H])
    og = jax.nn.sigmoid(gates[:, 3 * _H:])
    c = fg * c0 + ig * gg
    h = og * jnp.tanh(c)

    comm = jnp.dot(h, W_me_ref[...], preferred_element_type=f32) + b_me_ref[...]

    row_env = jax.lax.broadcasted_iota(jnp.int32, (_R, _R), 0) // _N
    col_env = jax.lax.broadcasted_iota(jnp.int32, (_R, _R), 1) // _N
    same_env = row_env == col_env

    def gat(cin, W, a_vec, heads, dout):
        Wh = jnp.dot(cin, W, preferred_element_type=f32)
        outs = []
        for hd in range(heads):
            Whh = Wh[:, hd * dout:(hd + 1) * dout]
            src = jnp.sum(Whh * a_vec[hd, :dout], axis=1, keepdims=True)
            dst = jnp.sum(Whh * a_vec[hd, dout:], axis=1, keepdims=True)
            e = src + jnp.transpose(dst)
            e = jnp.where(e >= 0, e, 0.2 * e)
            e = jnp.where(same_env, e, -1e9)
            e = e - jnp.max(e, axis=1, keepdims=True)
            w = jnp.exp(e)
            attn = w / jnp.sum(w, axis=1, keepdims=True)
            outs.append(jnp.dot(attn, Whh, preferred_element_type=f32))
        return outs[0] if heads == 1 else jnp.concatenate(outs, axis=1)

    c1 = gat(comm, W_g1_ref[...], a_g1_ref[...], 4, 32)
    c1 = jnp.where(c1 > 0, c1, jnp.exp(jnp.minimum(c1, 0.0)) - 1.0)
    c2 = gat(c1, W_g2_ref[...], a_g2_ref[...], 1, 128)

    comm_out = jnp.dot(c2, W_md_ref[...], preferred_element_type=f32) + b_md_ref[...]
    feat = jnp.concatenate([h, comm_out], axis=1)
    out_ref[...] = (jnp.dot(feat, W_act_ref[...], preferred_element_type=f32)
                    + b_act_ref[...])


def kernel(obs, rnn_states, masks, W_obs, b_obs, W_ih, W_hh, b_ih, b_hh,
           W_me, b_me, W_g1, a_g1, W_g2, a_g2, W_md, b_md, W_act, b_act):
    BN = _B * _N
    rnn2 = rnn_states.reshape(BN, 2 * _H)
    b_gates = (b_ih + b_hh).reshape(1, 4 * _H)

    def full(shape):
        return pl.BlockSpec(shape, lambda i: (0,) * len(shape))

    grid = (_B // _BB,)
    return pl.pallas_call(
        _fused,
        grid=grid,
        in_specs=[
            pl.BlockSpec((_R, 128), lambda i: (i, 0)),
            pl.BlockSpec((_R, 2 * _H), lambda i: (i, 0)),
            pl.BlockSpec((_R, 1), lambda i: (i, 0)),
            full((128, _H)),
            full((1, _H)),
            full((_H, 4 * _H)),
            full((_H, 4 * _H)),
            full((1, 4 * _H)),
            full((_H, _H)),
            full((1, _H)),
            full((_H, 128)),
            full((4, 64)),
            full((128, 128)),
            full((1, 256)),
            full((_H, _H)),
            full((1, _H)),
            full((2 * _H, _ACT)),
            full((1, _ACT)),
        ],
        out_specs=pl.BlockSpec((_R, _ACT), lambda i: (i, 0)),
        out_shape=jax.ShapeDtypeStruct((BN, _ACT), jnp.float32),
        compiler_params=pltpu.CompilerParams(
            dimension_semantics=("arbitrary",)),
    )(obs, rnn2, masks, W_obs, b_obs.reshape(1, _H), W_ih, W_hh, b_gates,
      W_me, b_me.reshape(1, _H), W_g1, a_g1, W_g2, a_g2, W_md,
      b_md.reshape(1, _H), W_act, b_act.reshape(1, _ACT))


# speedup vs baseline: 1.4982x; 1.4982x over previous
import jax
import jax.numpy as jnp
from jax.experimental import pallas as pl
from jax.experimental.pallas import tpu as pltpu

_B, _N, _H = 1024, 50, 128
_ACT = 19
_BB = 8
_R = _BB * _N


def _fused(obs_ref, rnn_ref, masks_ref, W_obs_ref, b_obs_ref, W_ih_ref,
           W_hh_ref, b_gates_ref, W_me_ref, b_me_ref, W_g1_ref, A1_ref,
           W_g2_ref, A2_ref, W_md_ref, b_md_ref, W_act_ref, b_act_ref,
           out_ref):
    f32 = jnp.float32

    x = jnp.tanh(jnp.dot(obs_ref[...], W_obs_ref[...],
                         preferred_element_type=f32) + b_obs_ref[...])
    m = masks_ref[...]
    h0 = rnn_ref[:, :_H] * m
    c0 = rnn_ref[:, _H:] * m
    gates = (jnp.dot(x, W_ih_ref[...], preferred_element_type=f32)
             + jnp.dot(h0, W_hh_ref[...], preferred_element_type=f32)
             + b_gates_ref[...])
    ig = jax.nn.sigmoid(gates[:, :_H])
    fg = jax.nn.sigmoid(gates[:, _H:2 * _H])
    gg = jnp.tanh(gates[:, 2 * _H:3 * _H])
    og = jax.nn.sigmoid(gates[:, 3 * _H:])
    c = fg * c0 + ig * gg
    h = og * jnp.tanh(c)

    comm = jnp.dot(h, W_me_ref[...], preferred_element_type=f32) + b_me_ref[...]

    row_env = jax.lax.broadcasted_iota(jnp.int32, (_R, _R), 0) // _N
    col_env = jax.lax.broadcasted_iota(jnp.int32, (_R, _R), 1) // _N
    neg_mask = jnp.where(row_env == col_env, 0.0, -1e9).astype(f32)

    def gat(cin, W, A, heads, dout):
        Wh = jnp.dot(cin, W, preferred_element_type=f32)
        sd = jnp.dot(Wh, A, preferred_element_type=f32)
        src_all = sd[:, :heads]
        dstT = jnp.transpose(sd[:, heads:])
        outs = []
        for hd in range(heads):
            e = src_all[:, hd:hd + 1] + dstT[hd:hd + 1, :]
            e = jnp.maximum(e, 0.2 * e) + neg_mask
            w = jnp.exp(e - jnp.max(e, axis=1, keepdims=True))
            num = jnp.dot(w, Wh[:, hd * dout:(hd + 1) * dout],
                          preferred_element_type=f32)
            outs.append(num / jnp.sum(w, axis=1, keepdims=True))
        return outs[0] if heads == 1 else jnp.concatenate(outs, axis=1)

    c1 = gat(comm, W_g1_ref[...], A1_ref[...], 4, 32)
    c1 = jnp.where(c1 > 0, c1, jnp.exp(jnp.minimum(c1, 0.0)) - 1.0)
    c2 = gat(c1, W_g2_ref[...], A2_ref[...], 1, 128)

    comm_out = jnp.dot(c2, W_md_ref[...], preferred_element_type=f32) + b_md_ref[...]
    feat = jnp.concatenate([h, comm_out], axis=1)
    out_ref[...] = (jnp.dot(feat, W_act_ref[...], preferred_element_type=f32)
                    + b_act_ref[...])


def kernel(obs, rnn_states, masks, W_obs, b_obs, W_ih, W_hh, b_ih, b_hh,
           W_me, b_me, W_g1, a_g1, W_g2, a_g2, W_md, b_md, W_act, b_act):
    BN = _B * _N
    rnn2 = rnn_states.reshape(BN, 2 * _H)
    b_gates = (b_ih + b_hh).reshape(1, 4 * _H)

    eye4 = jnp.eye(4, dtype=jnp.float32)
    A1s = (a_g1[:, :32, None] * eye4[:, None, :]).reshape(128, 4)
    A1d = (a_g1[:, 32:, None] * eye4[:, None, :]).reshape(128, 4)
    A1 = jnp.concatenate([A1s, A1d], axis=1)
    A2 = jnp.stack([a_g2[0, :128], a_g2[0, 128:]], axis=1)

    def full(shape):
        return pl.BlockSpec(shape, lambda i: (0,) * len(shape))

    grid = (_B // _BB,)
    return pl.pallas_call(
        _fused,
        grid=grid,
        in_specs=[
            pl.BlockSpec((_R, 128), lambda i: (i, 0)),
            pl.BlockSpec((_R, 2 * _H), lambda i: (i, 0)),
            pl.BlockSpec((_R, 1), lambda i: (i, 0)),
            full((128, _H)),
            full((1, _H)),
            full((_H, 4 * _H)),
            full((_H, 4 * _H)),
            full((1, 4 * _H)),
            full((_H, _H)),
            full((1, _H)),
            full((_H, 128)),
            full((128, 8)),
            full((128, 128)),
            full((128, 2)),
            full((_H, _H)),
            full((1, _H)),
            full((2 * _H, _ACT)),
            full((1, _ACT)),
        ],
        out_specs=pl.BlockSpec((_R, _ACT), lambda i: (i, 0)),
        out_shape=jax.ShapeDtypeStruct((BN, _ACT), jnp.float32),
        compiler_params=pltpu.CompilerParams(
            dimension_semantics=("arbitrary",)),
    )(obs, rnn2, masks, W_obs, b_obs.reshape(1, _H), W_ih, W_hh, b_gates,
      W_me, b_me.reshape(1, _H), W_g1, A1, W_g2, A2, W_md,
      b_md.reshape(1, _H), W_act, b_act.reshape(1, _ACT))
